# baseline (speedup 1.0000x reference)
import functools
import math

import jax
import jax.numpy as jnp
from jax import lax
from jax.experimental import pallas as pl
from jax.experimental.pallas import tpu as pltpu
from jax.experimental.pallas import tpu_sc as plsc

_VOCAB = 100000
_D = 64
_SCALE = math.sqrt(_D)

_NC = 2
_NS = 16
_NW = _NC * _NS

_BATCH = 4096
_T = 200
_BB = _BATCH // _NW
_NG = 4
_NT = 2


def _gather_body(xt_hbm, table_hbm, out_hbm, idx_v, gbuf, tbuf, gsem, wsem):
    wid = lax.axis_index("s") * _NC + lax.axis_index("c")

    pltpu.sync_copy(xt_hbm.at[:, pl.ds(wid * _BB, _BB)], idx_v)

    lane = lax.iota(jnp.int32, 16)

    def _issue_gather(t, g):
        pltpu.async_copy(table_hbm.at[idx_v.at[t]], gbuf.at[g], gsem)

    def _wait_gather(t, g):
        pltpu.make_async_copy(table_hbm.at[idx_v.at[t]], gbuf.at[g], gsem).wait()

    def _issue_write(t, p):
        pltpu.async_copy(tbuf.at[p], out_hbm.at[t, :, wid], wsem)

    def _wait_write(t, p):
        pltpu.make_async_copy(tbuf.at[p], out_hbm.at[t, :, wid], wsem).wait()

    def _transpose(g, p):
        @pl.loop(0, _BB // 16)
        def _bg(bg):
            row = lane + bg * 16
            for d in range(_D):
                col = jnp.full((16,), d, jnp.int32)
                v = plsc.load_gather(gbuf.at[g], [row, col])
                tbuf[p, d // 8, d % 8, pl.ds(bg * 16, 16)] = v * _SCALE

    for g in range(_NG):
        _issue_gather(g, g)

    @pl.loop(0, _T // _NG)
    def _outer(o):
        t0 = o * _NG
        for g in range(_NG):
            t = t0 + g
            p = g % _NT
            _wait_gather(t, g)

            @pl.when(t >= _NT)
            def _():
                _wait_write(jnp.maximum(t - _NT, 0), p)

            _transpose(g, p)
            _issue_write(t, p)

            @pl.when(t + _NG < _T)
            def _():
                _issue_gather(t + _NG, g)

    for g in range(_NG - _NT, _NG):
        _wait_write(_T - _NG + g, g % _NT)


@functools.partial(
    pl.kernel,
    out_type=jax.ShapeDtypeStruct((_T, _D // 8, _NW, 8, 128), jnp.float32),
    mesh=plsc.VectorSubcoreMesh(core_axis_name="c", subcore_axis_name="s"),
    compiler_params=pltpu.CompilerParams(
        use_tc_tiling_on_sc=False, needs_layout_passes=False
    ),
    scratch_types=[
        pltpu.VMEM((_T, _BB), jnp.int32),
        pltpu.VMEM((_NG, _BB, _D), jnp.float32),
        pltpu.VMEM((_NT, _D // 8, 8, 128), jnp.float32),
        pltpu.SemaphoreType.DMA,
        pltpu.SemaphoreType.DMA,
    ],
)
def _gather_rows(xt_hbm, table_hbm, out_hbm, idx_v, gbuf, tbuf, gsem, wsem):
    _gather_body(xt_hbm, table_hbm, out_hbm, idx_v, gbuf, tbuf, gsem, wsem)


def kernel(x, weight):
    out5 = _gather_rows(x.T, weight)
    return out5.transpose(2, 4, 0, 1, 3).reshape(_BATCH, _T, _D)


# speedup vs baseline: 6.4998x; 6.4998x over previous
import functools
import math

import jax
import jax.numpy as jnp
from jax import lax
from jax.experimental import pallas as pl
from jax.experimental.pallas import tpu as pltpu
from jax.experimental.pallas import tpu_sc as plsc

_VOCAB = 100000
_D = 64
_SCALE = math.sqrt(_D)

_NC = 2
_NS = 16
_NW = _NC * _NS

_BATCH = 4096
_T = 200
_BB = _BATCH // _NW
_NG = 4
_NT = 2


def _gather_body(xt_hbm, table_hbm, out_hbm, idx_v, gbuf, tbuf, gsem, wsem):
    wid = lax.axis_index("s") * _NC + lax.axis_index("c")

    pltpu.sync_copy(xt_hbm.at[:, pl.ds(wid * _BB, _BB)], idx_v)

    lane = lax.iota(jnp.int32, 16)

    def _issue_gather(t, g):
        pltpu.async_copy(table_hbm.at[idx_v.at[t]], gbuf.at[g], gsem)

    def _wait_gather(t, g):
        pltpu.make_async_copy(table_hbm.at[idx_v.at[t]], gbuf.at[g], gsem).wait()

    def _issue_write(t, p):
        pltpu.async_copy(tbuf.at[p], out_hbm.at[t, :, wid], wsem)

    def _wait_write(t, p):
        pltpu.make_async_copy(tbuf.at[p], out_hbm.at[t, :, wid], wsem).wait()

    nbg = _BB // 16
    zeros16 = jnp.zeros((16,), jnp.int32)

    def _transpose(g, p):
        @functools.partial(plsc.parallel_loop, 0, _D * nbg, unroll=8)
        def _k(k):
            d = k // nbg
            bg = k % nbg
            row = lane + bg * 16
            col = zeros16 + d
            v = plsc.load_gather(gbuf.at[g], [row, col])
            tbuf[p, d // 8, d % 8, pl.ds(bg * 16, 16)] = v * _SCALE

    for g in range(_NG):
        _issue_gather(g, g)

    @pl.loop(0, _T // _NG)
    def _outer(o):
        t0 = o * _NG
        for g in range(_NG):
            t = t0 + g
            p = g % _NT
            _wait_gather(t, g)

            @pl.when(t >= _NT)
            def _():
                _wait_write(jnp.maximum(t - _NT, 0), p)

            _transpose(g, p)
            _issue_write(t, p)

            @pl.when(t + _NG < _T)
            def _():
                _issue_gather(t + _NG, g)

    for g in range(_NG - _NT, _NG):
        _wait_write(_T - _NG + g, g % _NT)


@functools.partial(
    pl.kernel,
    out_type=jax.ShapeDtypeStruct((_T, _D // 8, _NW, 8, 128), jnp.float32),
    mesh=plsc.VectorSubcoreMesh(core_axis_name="c", subcore_axis_name="s"),
    compiler_params=pltpu.CompilerParams(
        use_tc_tiling_on_sc=False, needs_layout_passes=False
    ),
    scratch_types=[
        pltpu.VMEM((_T, _BB), jnp.int32),
        pltpu.VMEM((_NG, _BB, _D), jnp.float32),
        pltpu.VMEM((_NT, _D // 8, 8, 128), jnp.float32),
        pltpu.SemaphoreType.DMA,
        pltpu.SemaphoreType.DMA,
    ],
)
def _gather_rows(xt_hbm, table_hbm, out_hbm, idx_v, gbuf, tbuf, gsem, wsem):
    _gather_body(xt_hbm, table_hbm, out_hbm, idx_v, gbuf, tbuf, gsem, wsem)


def kernel(x, weight):
    out5 = _gather_rows(x.T, weight)
    return out5.transpose(2, 4, 0, 1, 3).reshape(_BATCH, _T, _D)
